# baseline (speedup 1.0000x reference)
import functools

import jax
import jax.numpy as jnp
from jax import lax
from jax.experimental import pallas as pl
from jax.experimental.pallas import tpu as pltpu
from jax.experimental.pallas import tpu_sc as plsc


def _pe_add_kernel(x_ref, emb_ref, o_ref):
    o_ref[...] = x_ref[...] + emb_ref[...]


def _kernel_tc(x, emb):
    B, S, D = x.shape
    BS = 2048
    grid = (S // BS, B)
    return pl.pallas_call(
        _pe_add_kernel,
        grid=grid,
        in_specs=[
            pl.BlockSpec((1, BS, D), lambda s, b: (b, s, 0)),
            pl.BlockSpec((BS, D), lambda s, b: (s, 0)),
        ],
        out_specs=pl.BlockSpec((1, BS, D), lambda s, b: (b, s, 0)),
        out_shape=jax.ShapeDtypeStruct((B, S, D), x.dtype),
    )(x, emb)


_NC, _NS, _L = 2, 16, 16


def _make_sc_kernel(B, S, D):
    N = B * S
    NW = _NC * _NS
    rows_per = N // NW
    R = 32
    chunks = rows_per // R

    mesh = plsc.VectorSubcoreMesh(core_axis_name="c", subcore_axis_name="s")

    @functools.partial(
        pl.kernel,
        mesh=mesh,
        out_type=jax.ShapeDtypeStruct((N, D), jnp.float32),
        scratch_types=[
            pltpu.VMEM((R, D), jnp.float32),
            pltpu.VMEM((R, D), jnp.float32),
        ],
    )
    def sc_pe_add(x_hbm, emb_hbm, out_hbm, xbuf, ebuf):
        wid = lax.axis_index("s") * _NC + lax.axis_index("c")
        base = wid * rows_per
        so_base = lax.rem(base, S)

        def _chunk(c, _):
            row0 = base + c * R
            so = so_base + c * R
            pltpu.sync_copy(x_hbm.at[pl.ds(row0, R)], xbuf)
            pltpu.sync_copy(emb_hbm.at[pl.ds(so, R)], ebuf)

            def _row(r, carry):
                for i in range(D // _L):
                    sl = pl.ds(i * _L, _L)
                    plsc.addupdate(xbuf.at[r, sl], ebuf[r, sl])
                return carry

            lax.fori_loop(0, R, _row, None)
            pltpu.sync_copy(xbuf, out_hbm.at[pl.ds(row0, R)])
            return _

        lax.fori_loop(0, chunks, _chunk, None)

    return sc_pe_add


def _kernel_sc(x, emb):
    B, S, D = x.shape
    out = _make_sc_kernel(B, S, D)(x.reshape(B * S, D), emb)
    return out.reshape(B, S, D)


def kernel(x, emb):
    return _kernel_sc(x, emb)


# speedup vs baseline: 2.3554x; 2.3554x over previous
import functools

import jax
import jax.numpy as jnp
from jax import lax
from jax.experimental import pallas as pl
from jax.experimental.pallas import tpu as pltpu
from jax.experimental.pallas import tpu_sc as plsc


def _pe_add_kernel(x_ref, emb_ref, o_ref):
    o_ref[...] = x_ref[...] + emb_ref[...]


def _kernel_tc(x, emb):
    B, S, D = x.shape
    BS = 2048
    grid = (S // BS, B)
    return pl.pallas_call(
        _pe_add_kernel,
        grid=grid,
        in_specs=[
            pl.BlockSpec((1, BS, D), lambda s, b: (b, s, 0)),
            pl.BlockSpec((BS, D), lambda s, b: (s, 0)),
        ],
        out_specs=pl.BlockSpec((1, BS, D), lambda s, b: (b, s, 0)),
        out_shape=jax.ShapeDtypeStruct((B, S, D), x.dtype),
    )(x, emb)


_NC, _NS, _L = 2, 16, 16


def _make_sc_kernel(B, S, D):
    N = B * S
    NW = _NC * _NS
    rows_per = N // NW
    R = 8
    NBUF = 4
    chunks = rows_per // R

    mesh = plsc.VectorSubcoreMesh(core_axis_name="c", subcore_axis_name="s")

    @functools.partial(
        pl.kernel,
        mesh=mesh,
        out_type=jax.ShapeDtypeStruct((N, D), jnp.float32),
        scratch_types=[
            pltpu.VMEM((NBUF, R, D), jnp.float32),
            pltpu.VMEM((NBUF, R, D), jnp.float32),
        ] + [pltpu.SemaphoreType.DMA] * (2 * NBUF),
    )
    def sc_pe_add(x_hbm, emb_hbm, out_hbm, xb, eb, *sems):
        lsems, ssems = sems[:NBUF], sems[NBUF:]
        wid = lax.axis_index("s") * _NC + lax.axis_index("c")
        base = wid * rows_per
        so_base = lax.rem(base, S)

        def start_load(c, b):
            row0 = base + c * R
            so = so_base + c * R
            pltpu.async_copy(x_hbm.at[pl.ds(row0, R)], xb.at[b], lsems[b])
            pltpu.async_copy(emb_hbm.at[pl.ds(so, R)], eb.at[b], lsems[b])

        def wait_load(c, b):
            row0 = base + c * R
            so = so_base + c * R
            pltpu.make_async_copy(
                x_hbm.at[pl.ds(row0, R)], xb.at[b], lsems[b]).wait()
            pltpu.make_async_copy(
                emb_hbm.at[pl.ds(so, R)], eb.at[b], lsems[b]).wait()

        def start_store(c, b):
            row0 = base + c * R
            pltpu.async_copy(xb.at[b], out_hbm.at[pl.ds(row0, R)], ssems[b])

        def wait_store(b):
            pltpu.make_async_copy(
                xb.at[b], out_hbm.at[pl.ds(base, R)], ssems[b]).wait()

        for b in range(NBUF - 1):
            start_load(b, b)

        def outer(t, _):
            c0 = t * NBUF
            for b in range(NBUF):
                c = c0 + b
                cp = c + (NBUF - 1)
                bp = (b + NBUF - 1) % NBUF

                @pl.when(cp < chunks)
                def _prefetch():
                    @pl.when(cp >= NBUF)
                    def _drain():
                        wait_store(bp)
                    start_load(cp, bp)

                wait_load(c, b)

                def _row(r, carry):
                    for i in range(D // _L):
                        sl = pl.ds(i * _L, _L)
                        plsc.addupdate(xb.at[b, r, sl], eb[b, r, sl])
                    return carry

                lax.fori_loop(0, R, _row, None)
                start_store(c, b)
            return _

        lax.fori_loop(0, chunks // NBUF, outer, None)
        for b in range(NBUF):
            wait_store(b)

    return sc_pe_add


def _kernel_sc(x, emb):
    B, S, D = x.shape
    out = _make_sc_kernel(B, S, D)(x.reshape(B * S, D), emb)
    return out.reshape(B, S, D)


def kernel(x, emb):
    return _kernel_sc(x, emb)


# speedup vs baseline: 2.8623x; 1.2152x over previous
import functools

import jax
import jax.numpy as jnp
from jax import lax
from jax.experimental import pallas as pl
from jax.experimental.pallas import tpu as pltpu
from jax.experimental.pallas import tpu_sc as plsc


def _pe_add_kernel(x_ref, emb_ref, o_ref):
    o_ref[...] = x_ref[...] + emb_ref[...]


def _kernel_tc(x, emb):
    B, S, D = x.shape
    BS = 2048
    grid = (S // BS, B)
    return pl.pallas_call(
        _pe_add_kernel,
        grid=grid,
        in_specs=[
            pl.BlockSpec((1, BS, D), lambda s, b: (b, s, 0)),
            pl.BlockSpec((BS, D), lambda s, b: (s, 0)),
        ],
        out_specs=pl.BlockSpec((1, BS, D), lambda s, b: (b, s, 0)),
        out_shape=jax.ShapeDtypeStruct((B, S, D), x.dtype),
    )(x, emb)


_NC, _NS, _L = 2, 16, 16


def _make_sc_kernel(B, S, D):
    N = B * S
    NW = _NC * _NS
    seq_per = S // NW
    R = 8
    NBUF = 2
    chunks = seq_per // R

    mesh = plsc.VectorSubcoreMesh(core_axis_name="c", subcore_axis_name="s")

    @functools.partial(
        pl.kernel,
        mesh=mesh,
        out_type=jax.ShapeDtypeStruct((N, D), jnp.float32),
        scratch_types=[
            pltpu.VMEM((NBUF, B, R, D), jnp.float32),
            pltpu.VMEM((NBUF, R, D), jnp.float32),
        ] + [pltpu.SemaphoreType.DMA] * (2 * NBUF),
    )
    def sc_pe_add(x_hbm, emb_hbm, out_hbm, xb, eb, *sems):
        lsems, ssems = sems[:NBUF], sems[NBUF:]
        wid = lax.axis_index("s") * _NC + lax.axis_index("c")
        seq0 = wid * seq_per

        def start_load(c, b):
            so = seq0 + c * R
            for bi in range(B):
                pltpu.async_copy(
                    x_hbm.at[pl.ds(bi * S + so, R)], xb.at[b, bi], lsems[b])
            pltpu.async_copy(emb_hbm.at[pl.ds(so, R)], eb.at[b], lsems[b])

        def wait_load(c, b):
            so = seq0 + c * R
            for bi in range(B):
                pltpu.make_async_copy(
                    x_hbm.at[pl.ds(bi * S + so, R)], xb.at[b, bi],
                    lsems[b]).wait()
            pltpu.make_async_copy(
                emb_hbm.at[pl.ds(so, R)], eb.at[b], lsems[b]).wait()

        def start_store(c, b):
            so = seq0 + c * R
            for bi in range(B):
                pltpu.async_copy(
                    xb.at[b, bi], out_hbm.at[pl.ds(bi * S + so, R)], ssems[b])

        def wait_store(b):
            for bi in range(B):
                pltpu.make_async_copy(
                    xb.at[b, bi], out_hbm.at[pl.ds(seq0, R)], ssems[b]).wait()

        for b in range(NBUF - 1):
            start_load(b, b)

        def outer(t, _):
            c0 = t * NBUF
            for b in range(NBUF):
                c = c0 + b
                cp = c + (NBUF - 1)
                bp = (b + NBUF - 1) % NBUF

                @pl.when(cp < chunks)
                def _prefetch():
                    @pl.when(cp >= NBUF)
                    def _drain():
                        wait_store(bp)
                    start_load(cp, bp)

                wait_load(c, b)

                def _row(r, carry):
                    for i in range(D // _L):
                        sl = pl.ds(i * _L, _L)
                        ev = eb[b, r, sl]
                        for bi in range(B):
                            plsc.addupdate(xb.at[b, bi, r, sl], ev)
                    return carry

                lax.fori_loop(0, R, _row, None)
                start_store(c, b)
            return _

        lax.fori_loop(0, chunks // NBUF, outer, None)
        for b in range(NBUF):
            wait_store(b)

    return sc_pe_add


def _kernel_sc(x, emb):
    B, S, D = x.shape
    out = _make_sc_kernel(B, S, D)(x.reshape(B * S, D), emb)
    return out.reshape(B, S, D)


def kernel(x, emb):
    return _kernel_sc(x, emb)
